# baseline (speedup 1.0000x reference)
import jax
import jax.numpy as jnp
from jax.experimental import pallas as pl

B = 64
SEQ = 96
NN = 325
E = 2600
OUT = 24
HEADS = 4
H1 = 64
H2 = 128
NT = 5


def _count_body(src_ref, dst_ref, c_ref):
    iota = jax.lax.broadcasted_iota(jnp.int32, (E, NN), 1)
    s_onehot = (src_ref[...] == iota).astype(jnp.float32)
    d_onehot = (dst_ref[...] == iota).astype(jnp.float32)
    c = jax.lax.dot_general(d_onehot, s_onehot, (((0,), (0,)), ((), ())),
                            preferred_element_type=jnp.float32)
    r = jax.lax.broadcasted_iota(jnp.int32, (NN, NN), 0)
    col = jax.lax.broadcasted_iota(jnp.int32, (NN, NN), 1)
    c_ref[...] = c + (r == col).astype(jnp.float32)


def _gat_layer(h, as_bd, ad_bd, cmat, oc):
    als_t = jax.lax.dot_general(as_bd, h, (((0,), (1,)), ((), ())),
                                preferred_element_type=jnp.float32)
    ald = jax.lax.dot_general(h, ad_bd, (((1,), (0,)), ((), ())),
                              preferred_element_type=jnp.float32)
    ones_col = jnp.ones((NN, 1), dtype=jnp.float32)
    ones_row = jnp.ones((1, NN), dtype=jnp.float32)
    acc = jnp.zeros((NN, oc), dtype=jnp.float32)
    for k in range(HEADS):
        row = als_t[k:k + 1, :]
        col = ald[:, k:k + 1]
        t = col + jnp.max(row)
        bd = jnp.maximum(t, 0.2 * t)
        u = jnp.concatenate([col - bd, 0.2 * col - bd], axis=1)
        x1 = jax.lax.dot_general(
            jnp.concatenate([u[:, 0:1], ones_col], axis=1),
            jnp.concatenate([ones_row, row], axis=0),
            (((1,), (0,)), ((), ())), preferred_element_type=jnp.float32,
            precision=jax.lax.Precision.HIGHEST)
        y1 = jax.lax.dot_general(
            jnp.concatenate([u[:, 1:2], ones_col], axis=1),
            jnp.concatenate([ones_row, 0.2 * row], axis=0),
            (((1,), (0,)), ((), ())), preferred_element_type=jnp.float32,
            precision=jax.lax.Precision.HIGHEST)
        p = cmat * jnp.exp(jnp.maximum(x1, y1))
        den = p.sum(axis=1, keepdims=True)
        rden = 1.0 / (den + 1e-16)
        hh = h[:, k * oc:(k + 1) * oc]
        num = jax.lax.dot_general(p, hh, (((1,), (0,)), ((), ())),
                                  preferred_element_type=jnp.float32)
        acc = acc + num * rden
    return acc * (1.0 / HEADS)


def _gat_body(x_ref, w1_ref, as1_ref, ad1_ref, b1_ref, w2_ref, as2_ref,
              ad2_ref, b2_ref, c_ref, out_ref):
    xb = x_ref[0]
    cmat = c_ref[...]

    h = jax.lax.dot_general(xb, w1_ref[...], (((0,), (0,)), ((), ())),
                            preferred_element_type=jnp.float32)
    o1 = _gat_layer(h, as1_ref[...], ad1_ref[...], cmat, H1) + b1_ref[...]
    o1 = jnp.where(o1 > 0, o1, jnp.exp(jnp.minimum(o1, 0.0)) - 1.0)

    h = jax.lax.dot_general(o1, w2_ref[...], (((1,), (0,)), ((), ())),
                            preferred_element_type=jnp.float32)
    out_ref[0] = _gat_layer(h, as2_ref[...], ad2_ref[...], cmat, H2) \
        + b2_ref[...]


def _mlp_body(g_ref, w1_ref, b1_ref, w2_ref, b2_ref, out_ref):
    for i in range(NT):
        g = g_ref[i]
        t = jax.lax.dot_general(g, w1_ref[i], (((1,), (0,)), ((), ())),
                                preferred_element_type=jnp.float32)
        t = jnp.maximum(t + b1_ref[i], 0.0)
        o = jax.lax.dot_general(t, w2_ref[i], (((1,), (0,)), ((), ())),
                                preferred_element_type=jnp.float32)
        out_ref[i] = o + b2_ref[i]


def _blockdiag(a):
    heads, oc = a.shape
    eye = jnp.eye(heads, dtype=a.dtype)
    return (a[:, :, None] * eye[:, None, :]).reshape(heads * oc, heads)


def kernel(x, edge_index, W1, a_s1, a_d1, b1, W2, a_s2, a_d2, b2,
           fW1, fb1, fW2, fb2):
    src = edge_index[0].reshape(E, 1)
    dst = edge_index[1].reshape(E, 1)

    cmat = pl.pallas_call(
        _count_body,
        out_shape=jax.ShapeDtypeStruct((NN, NN), jnp.float32),
    )(src, dst)

    full = lambda shp: pl.BlockSpec(shp, lambda b: (0,) * len(shp))
    gat = pl.pallas_call(
        _gat_body,
        grid=(B,),
        in_specs=[
            pl.BlockSpec((1, SEQ, NN), lambda b: (b, 0, 0)),
            full((SEQ, HEADS * H1)),
            full((HEADS * H1, HEADS)), full((HEADS * H1, HEADS)),
            full((1, H1)),
            full((H1, HEADS * H2)),
            full((HEADS * H2, HEADS)), full((HEADS * H2, HEADS)),
            full((1, H2)),
            full((NN, NN)),
        ],
        out_specs=pl.BlockSpec((1, NN, H2), lambda b: (b, 0, 0)),
        out_shape=jax.ShapeDtypeStruct((B, NN, H2), jnp.float32),
    )(x, W1, _blockdiag(a_s1), _blockdiag(a_d1), b1.reshape(1, H1),
      W2, _blockdiag(a_s2), _blockdiag(a_d2), b2.reshape(1, H2), cmat)

    g = jnp.transpose(gat, (1, 0, 2))
    pred = pl.pallas_call(
        _mlp_body,
        grid=(NN // NT,),
        in_specs=[
            pl.BlockSpec((NT, B, H2), lambda n: (n, 0, 0)),
            pl.BlockSpec((NT, H2, 64), lambda n: (n, 0, 0)),
            pl.BlockSpec((NT, 1, 64), lambda n: (n, 0, 0)),
            pl.BlockSpec((NT, 64, OUT), lambda n: (n, 0, 0)),
            pl.BlockSpec((NT, 1, OUT), lambda n: (n, 0, 0)),
        ],
        out_specs=pl.BlockSpec((NT, B, OUT), lambda n: (n, 0, 0)),
        out_shape=jax.ShapeDtypeStruct((NN, B, OUT), jnp.float32),
    )(g, fW1, fb1.reshape(NN, 1, 64), fW2, fb2.reshape(NN, 1, OUT))
    return pred


# speedup vs baseline: 2.6991x; 2.6991x over previous
import jax
import jax.numpy as jnp
from jax.experimental import pallas as pl

B = 64
SEQ = 96
NN = 325
E = 2600
OUT = 24
HEADS = 4
H1 = 64
H2 = 128
NT = 8


def _count_body(src_ref, dst_ref, c_ref):
    iota = jax.lax.broadcasted_iota(jnp.int32, (E, NN), 1)
    s_onehot = (src_ref[...] == iota).astype(jnp.float32)
    d_onehot = (dst_ref[...] == iota).astype(jnp.float32)
    c = jax.lax.dot_general(d_onehot, s_onehot, (((0,), (0,)), ((), ())),
                            preferred_element_type=jnp.float32)
    r = jax.lax.broadcasted_iota(jnp.int32, (NN, NN), 0)
    col = jax.lax.broadcasted_iota(jnp.int32, (NN, NN), 1)
    c_ref[...] = c + (r == col).astype(jnp.float32)


def _gat_layer(h, as_bd, ad_bd, cmat, oc):
    als_t = jax.lax.dot_general(as_bd, h, (((0,), (1,)), ((), ())),
                                preferred_element_type=jnp.float32)
    ald = jax.lax.dot_general(h, ad_bd, (((1,), (0,)), ((), ())),
                              preferred_element_type=jnp.float32)
    ones_col = jnp.ones((NN, 1), dtype=jnp.float32)
    ones_row = jnp.ones((1, NN), dtype=jnp.float32)
    acc = jnp.zeros((NN, oc), dtype=jnp.float32)
    for k in range(HEADS):
        row = als_t[k:k + 1, :]
        col = ald[:, k:k + 1]
        t = col + jnp.max(row)
        bd = jnp.maximum(t, 0.2 * t)
        e = col + row
        e = jnp.maximum(e, 0.2 * e)
        p = cmat * jnp.exp(e - bd)
        den = p.sum(axis=1, keepdims=True)
        rden = 1.0 / (den + 1e-16)
        hh = h[:, k * oc:(k + 1) * oc]
        num = jax.lax.dot_general(p, hh, (((1,), (0,)), ((), ())),
                                  preferred_element_type=jnp.float32)
        acc = acc + num * rden
    return acc * (1.0 / HEADS)


def _gat_body(x_ref, w1_ref, as1_ref, ad1_ref, b1_ref, w2_ref, as2_ref,
              ad2_ref, b2_ref, c_ref, out_ref):
    xb = x_ref[0]
    cmat = c_ref[...]

    h = jax.lax.dot_general(xb, w1_ref[...], (((0,), (0,)), ((), ())),
                            preferred_element_type=jnp.float32)
    o1 = _gat_layer(h, as1_ref[...], ad1_ref[...], cmat, H1) + b1_ref[...]
    o1 = jnp.where(o1 > 0, o1, jnp.exp(jnp.minimum(o1, 0.0)) - 1.0)

    h = jax.lax.dot_general(o1, w2_ref[...], (((1,), (0,)), ((), ())),
                            preferred_element_type=jnp.float32)
    out_ref[0] = _gat_layer(h, as2_ref[...], ad2_ref[...], cmat, H2) \
        + b2_ref[...]


def _mlp_body(g_ref, w1_ref, b1_ref, w2_ref, b2_ref, out_ref):
    for i in range(NT):
        g = g_ref[:, i, :]
        t = jax.lax.dot_general(g, w1_ref[i], (((1,), (0,)), ((), ())),
                                preferred_element_type=jnp.float32)
        t = jnp.maximum(t + b1_ref[i], 0.0)
        o = jax.lax.dot_general(t, w2_ref[i], (((1,), (0,)), ((), ())),
                                preferred_element_type=jnp.float32)
        out_ref[i] = o + b2_ref[i]


def _blockdiag(a):
    heads, oc = a.shape
    eye = jnp.eye(heads, dtype=a.dtype)
    return (a[:, :, None] * eye[:, None, :]).reshape(heads * oc, heads)


def kernel(x, edge_index, W1, a_s1, a_d1, b1, W2, a_s2, a_d2, b2,
           fW1, fb1, fW2, fb2):
    src = edge_index[0].reshape(E, 1)
    dst = edge_index[1].reshape(E, 1)

    cmat = pl.pallas_call(
        _count_body,
        out_shape=jax.ShapeDtypeStruct((NN, NN), jnp.float32),
    )(src, dst)

    full = lambda shp: pl.BlockSpec(shp, lambda b: (0,) * len(shp))
    gat = pl.pallas_call(
        _gat_body,
        grid=(B,),
        in_specs=[
            pl.BlockSpec((1, SEQ, NN), lambda b: (b, 0, 0)),
            full((SEQ, HEADS * H1)),
            full((HEADS * H1, HEADS)), full((HEADS * H1, HEADS)),
            full((1, H1)),
            full((H1, HEADS * H2)),
            full((HEADS * H2, HEADS)), full((HEADS * H2, HEADS)),
            full((1, H2)),
            full((NN, NN)),
        ],
        out_specs=pl.BlockSpec((1, NN, H2), lambda b: (b, 0, 0)),
        out_shape=jax.ShapeDtypeStruct((B, NN, H2), jnp.float32),
    )(x, W1, _blockdiag(a_s1), _blockdiag(a_d1), b1.reshape(1, H1),
      W2, _blockdiag(a_s2), _blockdiag(a_d2), b2.reshape(1, H2), cmat)

    pred = pl.pallas_call(
        _mlp_body,
        grid=((NN + NT - 1) // NT,),
        in_specs=[
            pl.BlockSpec((B, NT, H2), lambda n: (0, n, 0)),
            pl.BlockSpec((NT, H2, 64), lambda n: (n, 0, 0)),
            pl.BlockSpec((NT, 1, 64), lambda n: (n, 0, 0)),
            pl.BlockSpec((NT, 64, OUT), lambda n: (n, 0, 0)),
            pl.BlockSpec((NT, 1, OUT), lambda n: (n, 0, 0)),
        ],
        out_specs=pl.BlockSpec((NT, B, OUT), lambda n: (n, 0, 0)),
        out_shape=jax.ShapeDtypeStruct((NN, B, OUT), jnp.float32),
    )(gat, fW1, fb1.reshape(NN, 1, 64), fW2, fb2.reshape(NN, 1, OUT))
    return pred
